# baseline (speedup 1.0000x reference)
import functools

import jax
import jax.numpy as jnp
from jax import lax
from jax.experimental import pallas as pl
from jax.experimental.pallas import tpu as pltpu
from jax.experimental.pallas import tpu_sc as plsc

B = 16384
NEG = 20
D = 64
L = 16
NC = 2
NS = 16
NW = NC * NS
BPW = B // NW
CHUNK = 64
NCHUNK = BPW // CHUNK
NIDX = CHUNK * NEG // 128


def _sc_dots(center_h, outside_h, neg2_h, wc_h, wo_h, uo_out, uk_out,
             idx_c, idx_o, idx_n, c_rows, o_rows, n_rows, uo_buf, uk_buf,
             sem):
    wid = lax.axis_index("s") * NC + lax.axis_index("c")
    base = wid * BPW
    pltpu.sync_copy(center_h.at[pl.ds(base, BPW)], idx_c)
    pltpu.sync_copy(outside_h.at[pl.ds(base, BPW)], idx_o)
    pltpu.sync_copy(neg2_h.at[pl.ds(wid * (NCHUNK * NIDX), NCHUNK * NIDX)],
                    idx_n)

    def chunk_body(t, _):
        cps = [pltpu.async_copy(wc_h.at[idx_c.at[pl.ds(t * CHUNK, CHUNK)]],
                                c_rows, sem),
               pltpu.async_copy(wo_h.at[idx_o.at[pl.ds(t * CHUNK, CHUNK)]],
                                o_rows, sem)]
        for j in range(NIDX):
            cps.append(pltpu.async_copy(wo_h.at[idx_n.at[t * NIDX + j]],
                                        n_rows.at[pl.ds(j * 128, 128)], sem))
        for cp in cps:
            cp.wait()

        def item_body(m, _):
            p = jnp.zeros((L,), jnp.float32)
            q = jnp.zeros((L,), jnp.float32)
            for j in range(D // L):
                cj = c_rows[m, pl.ds(j * L, L)]
                oj = o_rows[m, pl.ds(j * L, L)]
                sj = n_rows[m * NEG, pl.ds(j * L, L)]
                for k in range(1, NEG):
                    sj = sj + n_rows[m * NEG + k, pl.ds(j * L, L)]
                p = p + cj * oj
                q = q + cj * sj
            uo_buf[pl.ds((t * CHUNK + m) * L, L)] = p
            uk_buf[pl.ds((t * CHUNK + m) * L, L)] = -q
            return 0

        lax.fori_loop(0, CHUNK, item_body, 0)
        return 0

    lax.fori_loop(0, NCHUNK, chunk_body, 0)
    pltpu.sync_copy(uo_buf, uo_out.at[pl.ds(base * L, BPW * L)])
    pltpu.sync_copy(uk_buf, uk_out.at[pl.ds(base * L, BPW * L)])


@functools.partial(
    pl.kernel,
    mesh=plsc.VectorSubcoreMesh(core_axis_name="c", subcore_axis_name="s"),
    compiler_params=pltpu.CompilerParams(use_tc_tiling_on_sc=False),
    out_type=[jax.ShapeDtypeStruct((B * L,), jnp.float32),
              jax.ShapeDtypeStruct((B * L,), jnp.float32)],
    scratch_types=[
        pltpu.VMEM((BPW,), jnp.int32),
        pltpu.VMEM((BPW,), jnp.int32),
        pltpu.VMEM((NCHUNK * NIDX, 128), jnp.int32),
        pltpu.VMEM((CHUNK, D), jnp.float32),
        pltpu.VMEM((CHUNK, D), jnp.float32),
        pltpu.VMEM((CHUNK * NEG, D), jnp.float32),
        pltpu.VMEM((BPW * L,), jnp.float32),
        pltpu.VMEM((BPW * L,), jnp.float32),
        pltpu.SemaphoreType.DMA,
    ],
)
def _sc_kernel(center_h, outside_h, neg2_h, wc_h, wo_h, uo_out, uk_out,
               idx_c, idx_o, idx_n, c_rows, o_rows, n_rows, uo_buf, uk_buf,
               sem):
    _sc_dots(center_h, outside_h, neg2_h, wc_h, wo_h, uo_out, uk_out,
             idx_c, idx_o, idx_n, c_rows, o_rows, n_rows, uo_buf, uk_buf,
             sem)


def _loss_body(uo_ref, uk_ref, out_ref):
    a = uo_ref[...].reshape(B // 8, 8, L).sum(axis=-1)
    b = uk_ref[...].reshape(B // 8, 8, L).sum(axis=-1)

    def logsig(x):
        return jnp.minimum(x, 0.0) - jnp.log(1.0 + jnp.exp(-jnp.abs(x)))

    out_ref[...] = jnp.full((1, 1), -jnp.sum(logsig(a) + logsig(b)) / B)


def kernel(center, outside, negative, W_center, W_outside):
    center = center.reshape(B)
    outside = outside.reshape(B)
    neg2 = negative.reshape(B * NEG // 128, 128)
    uo, uk = _sc_kernel(center, outside, neg2, W_center, W_outside)
    loss = pl.pallas_call(
        _loss_body,
        out_shape=jax.ShapeDtypeStruct((1, 1), jnp.float32),
    )(uo.reshape(B // 8, 8 * L), uk.reshape(B // 8, 8 * L))
    return loss[0, 0]
